# baseline (speedup 1.0000x reference)
import functools

import jax
import jax.numpy as jnp
from jax import lax
from jax.experimental import pallas as pl
from jax.experimental.pallas import tpu as pltpu
from jax.experimental.pallas import tpu_sc as plsc

N = 10000
E = 320000
F = 128
BN = 1000

NC, NS = 2, 16
NW = NC * NS
NPAD = 10240
NPT = NPAD // NS
CB = 128
EROWS = 2560
EPAD = EROWS * CB
AGG_ROWS = EROWS // NW
DEG_ROWS = EROWS // NS
DEG_K = 8


def _rsqrt_newton(d):
    i = lax.bitcast_convert_type(d, jnp.int32)
    i = jnp.int32(0x5F3759DF) - lax.shift_right_logical(i, 1)
    y = lax.bitcast_convert_type(i, jnp.float32)
    for _ in range(3):
        y = y * (1.5 - 0.5 * d * y * y)
    return y


def _sc_body(src_hbm, dst_hbm, ew_hbm, x_hbm, aggp_hbm, dis_hbm,
             srcA, srcB, dstA, dstB, ewA, ewB, disA, disB,
             rows0, rows1, disv,
             agg_sh, deg_sh, dis_sh,
             sem_lA, sem_lB, sem_g0, sem_g1, sem_dA, sem_dB, sem_s0, sem_s1):
    cid = lax.axis_index("c")
    sid = lax.axis_index("s")
    wid = cid * NS + sid
    LAST = AGG_ROWS // 2 - 1
    DLAST = DEG_ROWS // 2 - 1

    def zrow(r, _):
        for j in range(F // 16):
            rows0[r, pl.ds(j * 16, 16)] = jnp.zeros((16,), jnp.float32)
        return 0
    lax.fori_loop(0, CB, zrow, 0)
    for q in range(NPT // CB):
        pltpu.sync_copy(rows0, agg_sh.at[pl.ds(sid * NPT + q * CB, CB)])
    def zdeg(r, _):
        disv[pl.ds(r * 16, 16)] = jnp.zeros((16,), jnp.float32)
        return 0
    lax.fori_loop(0, NPT // 16, zdeg, 0)
    pltpu.sync_copy(disv, deg_sh.at[pl.ds(sid * NPT, NPT)])
    plsc.subcore_barrier()

    def lin_load(cr, dbuf, ebuf, sem, with_src=None):
        pltpu.async_copy(dst_hbm.at[cr], dbuf, sem)
        pltpu.async_copy(ew_hbm.at[cr], ebuf, sem)
        if with_src is not None:
            pltpu.async_copy(src_hbm.at[cr], with_src, sem)
    def lin_wait(dbuf, ebuf, sem, with_src=None):
        pltpu.make_async_copy(dst_hbm.at[0], dbuf, sem).wait()
        pltpu.make_async_copy(ew_hbm.at[0], ebuf, sem).wait()
        if with_src is not None:
            pltpu.make_async_copy(src_hbm.at[0], with_src, sem).wait()
    def deg_scat(dbuf, ebuf, sem):
        pltpu.async_copy(ebuf, deg_sh.at[dbuf], sem, add=True)
    def deg_wait(ebuf, sem):
        pltpu.make_async_copy(ebuf, deg_sh.at[pl.ds(0, CB)], sem).wait()

    dbase = sid * DEG_ROWS
    lin_load(dbase, dstA, ewA, sem_lA)
    lin_wait(dstA, ewA, sem_lA)
    def deg_loop(k, _):
        c0 = dbase + 2 * k
        lin_load(c0 + 1, dstB, ewB, sem_lB)
        deg_scat(dstA, ewA, sem_s0)
        lin_wait(dstB, ewB, sem_lB)
        deg_wait(ewA, sem_s0)
        @pl.when(k < DLAST)
        def _():
            lin_load(c0 + 2, dstA, ewA, sem_lA)
        deg_scat(dstB, ewB, sem_s1)
        @pl.when(k < DLAST)
        def _():
            lin_wait(dstA, ewA, sem_lA)
        deg_wait(ewB, sem_s1)
        return 0
    lax.fori_loop(0, DEG_ROWS // 2, deg_loop, 0)
    plsc.subcore_barrier()

    pltpu.sync_copy(deg_sh.at[pl.ds(sid * NPT, NPT)], disv)
    def dis_loop(r, _):
        d = disv[pl.ds(r * 16, 16)] + 1.0
        disv[pl.ds(r * 16, 16)] = _rsqrt_newton(d)
        return 0
    lax.fori_loop(0, NPT // 16, dis_loop, 0)
    pltpu.sync_copy(disv, dis_sh.at[pl.ds(sid * NPT, NPT)])
    @pl.when(cid == 0)
    def _():
        pltpu.sync_copy(disv, dis_hbm.at[pl.ds(sid * NPT, NPT)])
    plsc.subcore_barrier()

    def scale(rows_b, ebuf, dbuf_dis):
        def grp(j, _):
            s16 = ebuf[pl.ds(j * 16, 16)] * dbuf_dis[pl.ds(j * 16, 16)]
            for i in range(16):
                s = s16[i]
                e = j * 16 + i
                for f_ in range(F // 16):
                    rows_b[e, pl.ds(f_ * 16, 16)] = rows_b[e, pl.ds(f_ * 16, 16)] * s
            return 0
        lax.fori_loop(0, CB // 16, grp, 0)

    def gather(sbuf, rows_b, sem):
        pltpu.async_copy(x_hbm.at[sbuf], rows_b, sem)
    def disgather(sbuf, dis_b, sem):
        pltpu.async_copy(dis_sh.at[sbuf], dis_b, sem)
    def wait_g(rows_b, sem):
        pltpu.make_async_copy(x_hbm.at[pl.ds(0, CB)], rows_b, sem).wait()
    def wait_dg(dis_b, sem):
        pltpu.make_async_copy(dis_sh.at[pl.ds(0, CB)], dis_b, sem).wait()
    def scatter(rows_b, dbuf, sem):
        pltpu.async_copy(rows_b, agg_sh.at[dbuf], sem, add=True)
    def wait_s(rows_b, sem):
        pltpu.make_async_copy(rows_b, agg_sh.at[pl.ds(0, CB)], sem).wait()

    base = wid * AGG_ROWS
    lin_load(base, dstA, ewA, sem_lA, with_src=srcA)
    lin_load(base + 1, dstB, ewB, sem_lB, with_src=srcB)
    lin_wait(dstA, ewA, sem_lA, with_src=srcA)
    gather(srcA, rows0, sem_g0)
    disgather(srcA, disA, sem_dA)
    def p2_loop(k, _):
        c0 = base + 2 * k
        lin_wait(dstB, ewB, sem_lB, with_src=srcB)
        gather(srcB, rows1, sem_g1)
        disgather(srcB, disB, sem_dB)
        wait_g(rows0, sem_g0)
        wait_dg(disA, sem_dA)
        scale(rows0, ewA, disA)
        scatter(rows0, dstA, sem_s0)
        wait_g(rows1, sem_g1)
        wait_dg(disB, sem_dB)
        scale(rows1, ewB, disB)
        wait_s(rows0, sem_s0)
        @pl.when(k < LAST)
        def _():
            lin_load(c0 + 2, dstA, ewA, sem_lA, with_src=srcA)
            lin_wait(dstA, ewA, sem_lA, with_src=srcA)
            gather(srcA, rows0, sem_g0)
            disgather(srcA, disA, sem_dA)
        scatter(rows1, dstB, sem_s1)
        wait_s(rows1, sem_s1)
        @pl.when(k < LAST)
        def _():
            lin_load(c0 + 3, dstB, ewB, sem_lB, with_src=srcB)
        return 0
    lax.fori_loop(0, AGG_ROWS // 2, p2_loop, 0)
    plsc.subcore_barrier()

    for q in range(NPT // CB):
        off = sid * NPT + q * CB
        pltpu.sync_copy(agg_sh.at[pl.ds(off, CB)], rows0)
        pltpu.sync_copy(rows0, aggp_hbm.at[cid].at[pl.ds(off, CB)])


def _sparse_stage(src2, dst2, ew2, x):
    mesh = plsc.VectorSubcoreMesh(core_axis_name="c", subcore_axis_name="s")
    f = pl.kernel(
        _sc_body,
        out_type=[jax.ShapeDtypeStruct((NC, NPAD, F), jnp.float32),
                  jax.ShapeDtypeStruct((NPAD,), jnp.float32)],
        mesh=mesh,
        scratch_types=[
            pltpu.VMEM((CB,), jnp.int32),
            pltpu.VMEM((CB,), jnp.int32),
            pltpu.VMEM((CB,), jnp.int32),
            pltpu.VMEM((CB,), jnp.int32),
            pltpu.VMEM((CB,), jnp.float32),
            pltpu.VMEM((CB,), jnp.float32),
            pltpu.VMEM((CB,), jnp.float32),
            pltpu.VMEM((CB,), jnp.float32),
            pltpu.VMEM((CB, F), jnp.float32),
            pltpu.VMEM((CB, F), jnp.float32),
            pltpu.VMEM((NPT,), jnp.float32),
            pltpu.VMEM_SHARED((NPAD, F), jnp.float32),
            pltpu.VMEM_SHARED((NPAD,), jnp.float32),
            pltpu.VMEM_SHARED((NPAD,), jnp.float32),
            pltpu.SemaphoreType.DMA,
            pltpu.SemaphoreType.DMA,
            pltpu.SemaphoreType.DMA,
            pltpu.SemaphoreType.DMA,
            pltpu.SemaphoreType.DMA,
            pltpu.SemaphoreType.DMA,
            pltpu.SemaphoreType.DMA,
            pltpu.SemaphoreType.DMA,
        ],
        compiler_params=pltpu.CompilerParams(needs_layout_passes=False),
    )
    return f(src2, dst2, ew2, x)


def _dense_body(dis_ref, x_ref, h_ref, agg_ref,
                wz_ref, bz_ref, wr_ref, br_ref, wh_ref, bh_ref,
                wlz_ref, blz_ref, wlr_ref, blr_ref, wlh_ref, blh_ref,
                wout_ref, bout_ref, y_ref, hn_ref):
    d = dis_ref[:]
    x = x_ref[:]
    h = h_ref[:]
    p = d * (agg_ref[0] + agg_ref[1]) + (d * d) * x

    def mm(a, b):
        return jax.lax.dot_general(a, b, (((1,), (0,)), ((), ())),
                                   preferred_element_type=jnp.float32)

    cz = mm(p, wz_ref[:]) + bz_ref[:]
    cr = mm(p, wr_ref[:]) + br_ref[:]
    ch = mm(p, wh_ref[:]) + bh_ref[:]

    z = jax.nn.sigmoid(mm(cz, wlz_ref[:F]) + mm(h, wlz_ref[F:]) + blz_ref[:])
    r = jax.nn.sigmoid(mm(cr, wlr_ref[:F]) + mm(h, wlr_ref[F:]) + blr_ref[:])
    ht = jnp.tanh(mm(ch, wlh_ref[:F]) + mm(h * r, wlh_ref[F:]) + blh_ref[:])
    hn = z * h + (1.0 - z) * ht
    hn_ref[:] = hn
    y_ref[:] = mm(jnp.maximum(hn, 0.0), wout_ref[:]) + bout_ref[:]


def _dense_stage(dis, x, h, agg, Wz, bz, Wr, br, Wh, bh,
                 Wlz, blz, Wlr, blr, Wlh, blh, Wout, bout):
    grid = (N // BN,)
    row_spec = pl.BlockSpec((BN, F), lambda i: (i, 0))
    full = pl.BlockSpec((2, BN, F), lambda i: (0, i, 0))
    w_spec = pl.BlockSpec((F, F), lambda i: (0, 0))
    wl_spec = pl.BlockSpec((2 * F, F), lambda i: (0, 0))
    b_spec = pl.BlockSpec((1, F), lambda i: (0, 0))
    return pl.pallas_call(
        _dense_body,
        grid=grid,
        in_specs=[
            pl.BlockSpec((BN, 1), lambda i: (i, 0)),
            row_spec, row_spec, full,
            w_spec, b_spec, w_spec, b_spec, w_spec, b_spec,
            wl_spec, b_spec, wl_spec, b_spec, wl_spec, b_spec,
            w_spec, b_spec,
        ],
        out_specs=[row_spec, row_spec],
        out_shape=[jax.ShapeDtypeStruct((N, F), jnp.float32),
                   jax.ShapeDtypeStruct((N, F), jnp.float32)],
    )(dis, x, h, agg,
      Wz, bz.reshape(1, F), Wr, br.reshape(1, F), Wh, bh.reshape(1, F),
      Wlz, blz.reshape(1, F), Wlr, blr.reshape(1, F), Wlh, blh.reshape(1, F),
      Wout, bout.reshape(1, F))


def kernel(g, node_feat, edge_weight, hidden_state, Wz, bz, Wr, br, Wh, bh,
           Wlz, blz, Wlr, blr, Wlh, blh, Wout, bout):
    src, dst = g[0], g[1]
    x = node_feat

    npadE = EPAD - E
    src2 = jnp.concatenate([src, jnp.zeros((npadE,), jnp.int32)]).reshape(EROWS, CB)
    dst2 = jnp.concatenate([dst, jnp.full((npadE,), NPAD - 1, jnp.int32)]).reshape(EROWS, CB)
    ew2 = jnp.concatenate([edge_weight, jnp.zeros((npadE,), jnp.float32)]).reshape(EROWS, CB)

    aggp, dis_pad = _sparse_stage(src2, dst2, ew2, x)
    agg2 = aggp[:, :N, :]
    dis = dis_pad[:N]

    y, hn = _dense_stage(dis.reshape(N, 1), x, hidden_state, agg2,
                         Wz, bz, Wr, br, Wh, bh,
                         Wlz, blz, Wlr, blr, Wlh, blh, Wout, bout)
    return (y, hn)


# speedup vs baseline: 1.0004x; 1.0004x over previous
import functools

import jax
import jax.numpy as jnp
from jax import lax
from jax.experimental import pallas as pl
from jax.experimental.pallas import tpu as pltpu
from jax.experimental.pallas import tpu_sc as plsc

N = 10000
E = 320000
F = 128
BN = 1000

NC, NS = 2, 16
NW = NC * NS
NPAD = 10240
NPT = NPAD // NS
CB = 128
EROWS = 2560
EPAD = EROWS * CB
AGG_ROWS = EROWS // NW
DEG_ROWS = EROWS // NS
DEG_K = 8


def _rsqrt_newton(d):
    i = lax.bitcast_convert_type(d, jnp.int32)
    i = jnp.int32(0x5F3759DF) - lax.shift_right_logical(i, 1)
    y = lax.bitcast_convert_type(i, jnp.float32)
    for _ in range(3):
        y = y * (1.5 - 0.5 * d * y * y)
    return y


def _sc_body(src_hbm, dst_hbm, ew_hbm, x_hbm, aggp_hbm, dis_hbm,
             srcA, srcB, dstA, dstB, ewA, ewB, disA, disB,
             rows0, rows1, disv,
             agg_sh, deg_sh, dis_sh,
             sem_lA, sem_lB, sem_g0, sem_g1, sem_dA, sem_dB, sem_s0, sem_s1):
    cid = lax.axis_index("c")
    sid = lax.axis_index("s")
    wid = cid * NS + sid
    LAST = AGG_ROWS // 2 - 1
    DLAST = DEG_ROWS // 2 - 1

    def zrow(r, _):
        for j in range(F // 16):
            rows0[r, pl.ds(j * 16, 16)] = jnp.zeros((16,), jnp.float32)
        return 0
    lax.fori_loop(0, CB, zrow, 0)
    for q in range(NPT // CB):
        pltpu.sync_copy(rows0, agg_sh.at[pl.ds(sid * NPT + q * CB, CB)])
    def zdeg(r, _):
        disv[pl.ds(r * 16, 16)] = jnp.zeros((16,), jnp.float32)
        return 0
    lax.fori_loop(0, NPT // 16, zdeg, 0)
    pltpu.sync_copy(disv, deg_sh.at[pl.ds(sid * NPT, NPT)])
    plsc.subcore_barrier()

    def lin_load(cr, dbuf, ebuf, sem, with_src=None):
        pltpu.async_copy(dst_hbm.at[cr], dbuf, sem)
        pltpu.async_copy(ew_hbm.at[cr], ebuf, sem)
        if with_src is not None:
            pltpu.async_copy(src_hbm.at[cr], with_src, sem)
    def lin_wait(dbuf, ebuf, sem, with_src=None):
        pltpu.make_async_copy(dst_hbm.at[0], dbuf, sem).wait()
        pltpu.make_async_copy(ew_hbm.at[0], ebuf, sem).wait()
        if with_src is not None:
            pltpu.make_async_copy(src_hbm.at[0], with_src, sem).wait()
    def deg_scat(dbuf, ebuf, sem):
        pltpu.async_copy(ebuf, deg_sh.at[dbuf], sem, add=True)
    def deg_wait(ebuf, sem):
        pltpu.make_async_copy(ebuf, deg_sh.at[pl.ds(0, CB)], sem).wait()

    scope_deg = jax.named_scope("sc_deg")
    scope_deg.__enter__()
    dbase = sid * DEG_ROWS
    lin_load(dbase, dstA, ewA, sem_lA)
    lin_wait(dstA, ewA, sem_lA)
    def deg_loop(k, _):
        c0 = dbase + 2 * k
        lin_load(c0 + 1, dstB, ewB, sem_lB)
        deg_scat(dstA, ewA, sem_s0)
        lin_wait(dstB, ewB, sem_lB)
        deg_wait(ewA, sem_s0)
        @pl.when(k < DLAST)
        def _():
            lin_load(c0 + 2, dstA, ewA, sem_lA)
        deg_scat(dstB, ewB, sem_s1)
        @pl.when(k < DLAST)
        def _():
            lin_wait(dstA, ewA, sem_lA)
        deg_wait(ewB, sem_s1)
        return 0
    lax.fori_loop(0, DEG_ROWS // 2, deg_loop, 0)
    plsc.subcore_barrier()
    scope_deg.__exit__(None, None, None)

    pltpu.sync_copy(deg_sh.at[pl.ds(sid * NPT, NPT)], disv)
    def dis_loop(r, _):
        d = disv[pl.ds(r * 16, 16)] + 1.0
        disv[pl.ds(r * 16, 16)] = _rsqrt_newton(d)
        return 0
    lax.fori_loop(0, NPT // 16, dis_loop, 0)
    pltpu.sync_copy(disv, dis_sh.at[pl.ds(sid * NPT, NPT)])
    @pl.when(cid == 0)
    def _():
        pltpu.sync_copy(disv, dis_hbm.at[pl.ds(sid * NPT, NPT)])
    plsc.subcore_barrier()

    def scale(rows_b, ebuf, dbuf_dis):
        def grp(j, _):
            s16 = ebuf[pl.ds(j * 16, 16)] * dbuf_dis[pl.ds(j * 16, 16)]
            for i in range(16):
                s = s16[i]
                e = j * 16 + i
                for f_ in range(F // 16):
                    rows_b[e, pl.ds(f_ * 16, 16)] = rows_b[e, pl.ds(f_ * 16, 16)] * s
            return 0
        lax.fori_loop(0, CB // 16, grp, 0)

    def gather(sbuf, rows_b, sem):
        pltpu.async_copy(x_hbm.at[sbuf], rows_b, sem)
    def disgather(sbuf, dis_b, sem):
        pltpu.async_copy(dis_sh.at[sbuf], dis_b, sem)
    def wait_g(rows_b, sem):
        pltpu.make_async_copy(x_hbm.at[pl.ds(0, CB)], rows_b, sem).wait()
    def wait_dg(dis_b, sem):
        pltpu.make_async_copy(dis_sh.at[pl.ds(0, CB)], dis_b, sem).wait()
    def scatter(rows_b, dbuf, sem):
        pltpu.async_copy(rows_b, agg_sh.at[dbuf], sem, add=True)
    def wait_s(rows_b, sem):
        pltpu.make_async_copy(rows_b, agg_sh.at[pl.ds(0, CB)], sem).wait()

    scope_agg = jax.named_scope("sc_agg")
    scope_agg.__enter__()
    base = wid * AGG_ROWS
    lin_load(base, dstA, ewA, sem_lA, with_src=srcA)
    lin_load(base + 1, dstB, ewB, sem_lB, with_src=srcB)
    lin_wait(dstA, ewA, sem_lA, with_src=srcA)
    gather(srcA, rows0, sem_g0)
    disgather(srcA, disA, sem_dA)
    def p2_loop(k, _):
        c0 = base + 2 * k
        lin_wait(dstB, ewB, sem_lB, with_src=srcB)
        gather(srcB, rows1, sem_g1)
        disgather(srcB, disB, sem_dB)
        wait_g(rows0, sem_g0)
        wait_dg(disA, sem_dA)
        scale(rows0, ewA, disA)
        scatter(rows0, dstA, sem_s0)
        wait_g(rows1, sem_g1)
        wait_dg(disB, sem_dB)
        scale(rows1, ewB, disB)
        wait_s(rows0, sem_s0)
        @pl.when(k < LAST)
        def _():
            lin_load(c0 + 2, dstA, ewA, sem_lA, with_src=srcA)
            lin_wait(dstA, ewA, sem_lA, with_src=srcA)
            gather(srcA, rows0, sem_g0)
            disgather(srcA, disA, sem_dA)
        scatter(rows1, dstB, sem_s1)
        wait_s(rows1, sem_s1)
        @pl.when(k < LAST)
        def _():
            lin_load(c0 + 3, dstB, ewB, sem_lB, with_src=srcB)
        return 0
    lax.fori_loop(0, AGG_ROWS // 2, p2_loop, 0)
    plsc.subcore_barrier()
    scope_agg.__exit__(None, None, None)

    for q in range(NPT // CB):
        off = sid * NPT + q * CB
        pltpu.sync_copy(agg_sh.at[pl.ds(off, CB)], rows0)
        pltpu.sync_copy(rows0, aggp_hbm.at[cid].at[pl.ds(off, CB)])


def _sparse_stage(src2, dst2, ew2, x):
    mesh = plsc.VectorSubcoreMesh(core_axis_name="c", subcore_axis_name="s")
    f = pl.kernel(
        _sc_body,
        out_type=[jax.ShapeDtypeStruct((NC, NPAD, F), jnp.float32),
                  jax.ShapeDtypeStruct((NPAD,), jnp.float32)],
        mesh=mesh,
        scratch_types=[
            pltpu.VMEM((CB,), jnp.int32),
            pltpu.VMEM((CB,), jnp.int32),
            pltpu.VMEM((CB,), jnp.int32),
            pltpu.VMEM((CB,), jnp.int32),
            pltpu.VMEM((CB,), jnp.float32),
            pltpu.VMEM((CB,), jnp.float32),
            pltpu.VMEM((CB,), jnp.float32),
            pltpu.VMEM((CB,), jnp.float32),
            pltpu.VMEM((CB, F), jnp.float32),
            pltpu.VMEM((CB, F), jnp.float32),
            pltpu.VMEM((NPT,), jnp.float32),
            pltpu.VMEM_SHARED((NPAD, F), jnp.float32),
            pltpu.VMEM_SHARED((NPAD,), jnp.float32),
            pltpu.VMEM_SHARED((NPAD,), jnp.float32),
            pltpu.SemaphoreType.DMA,
            pltpu.SemaphoreType.DMA,
            pltpu.SemaphoreType.DMA,
            pltpu.SemaphoreType.DMA,
            pltpu.SemaphoreType.DMA,
            pltpu.SemaphoreType.DMA,
            pltpu.SemaphoreType.DMA,
            pltpu.SemaphoreType.DMA,
        ],
        compiler_params=pltpu.CompilerParams(needs_layout_passes=False),
    )
    return f(src2, dst2, ew2, x)


def _dense_body(dis_ref, x_ref, h_ref, agg_ref,
                wz_ref, bz_ref, wr_ref, br_ref, wh_ref, bh_ref,
                wlz_ref, blz_ref, wlr_ref, blr_ref, wlh_ref, blh_ref,
                wout_ref, bout_ref, y_ref, hn_ref):
    d = dis_ref[:]
    x = x_ref[:]
    h = h_ref[:]
    p = d * (agg_ref[0] + agg_ref[1]) + (d * d) * x

    def mm(a, b):
        return jax.lax.dot_general(a, b, (((1,), (0,)), ((), ())),
                                   preferred_element_type=jnp.float32)

    cz = mm(p, wz_ref[:]) + bz_ref[:]
    cr = mm(p, wr_ref[:]) + br_ref[:]
    ch = mm(p, wh_ref[:]) + bh_ref[:]

    z = jax.nn.sigmoid(mm(cz, wlz_ref[:F]) + mm(h, wlz_ref[F:]) + blz_ref[:])
    r = jax.nn.sigmoid(mm(cr, wlr_ref[:F]) + mm(h, wlr_ref[F:]) + blr_ref[:])
    ht = jnp.tanh(mm(ch, wlh_ref[:F]) + mm(h * r, wlh_ref[F:]) + blh_ref[:])
    hn = z * h + (1.0 - z) * ht
    hn_ref[:] = hn
    y_ref[:] = mm(jnp.maximum(hn, 0.0), wout_ref[:]) + bout_ref[:]


def _dense_stage(dis, x, h, agg, Wz, bz, Wr, br, Wh, bh,
                 Wlz, blz, Wlr, blr, Wlh, blh, Wout, bout):
    grid = (N // BN,)
    row_spec = pl.BlockSpec((BN, F), lambda i: (i, 0))
    full = pl.BlockSpec((2, BN, F), lambda i: (0, i, 0))
    w_spec = pl.BlockSpec((F, F), lambda i: (0, 0))
    wl_spec = pl.BlockSpec((2 * F, F), lambda i: (0, 0))
    b_spec = pl.BlockSpec((1, F), lambda i: (0, 0))
    return pl.pallas_call(
        _dense_body,
        grid=grid,
        in_specs=[
            pl.BlockSpec((BN, 1), lambda i: (i, 0)),
            row_spec, row_spec, full,
            w_spec, b_spec, w_spec, b_spec, w_spec, b_spec,
            wl_spec, b_spec, wl_spec, b_spec, wl_spec, b_spec,
            w_spec, b_spec,
        ],
        out_specs=[row_spec, row_spec],
        out_shape=[jax.ShapeDtypeStruct((N, F), jnp.float32),
                   jax.ShapeDtypeStruct((N, F), jnp.float32)],
    )(dis, x, h, agg,
      Wz, bz.reshape(1, F), Wr, br.reshape(1, F), Wh, bh.reshape(1, F),
      Wlz, blz.reshape(1, F), Wlr, blr.reshape(1, F), Wlh, blh.reshape(1, F),
      Wout, bout.reshape(1, F))


def kernel(g, node_feat, edge_weight, hidden_state, Wz, bz, Wr, br, Wh, bh,
           Wlz, blz, Wlr, blr, Wlh, blh, Wout, bout):
    src, dst = g[0], g[1]
    x = node_feat

    npadE = EPAD - E
    src2 = jnp.concatenate([src, jnp.zeros((npadE,), jnp.int32)]).reshape(EROWS, CB)
    dst2 = jnp.concatenate([dst, jnp.full((npadE,), NPAD - 1, jnp.int32)]).reshape(EROWS, CB)
    ew2 = jnp.concatenate([edge_weight, jnp.zeros((npadE,), jnp.float32)]).reshape(EROWS, CB)

    aggp, dis_pad = _sparse_stage(src2, dst2, ew2, x)
    agg2 = aggp[:, :N, :]
    dis = dis_pad[:N]

    y, hn = _dense_stage(dis.reshape(N, 1), x, hidden_state, agg2,
                         Wz, bz, Wr, br, Wh, bh,
                         Wlz, blz, Wlr, blr, Wlh, blh, Wout, bout)
    return (y, hn)


# speedup vs baseline: 1.2457x; 1.2453x over previous
import functools

import jax
import jax.numpy as jnp
from jax import lax
from jax.experimental import pallas as pl
from jax.experimental.pallas import tpu as pltpu
from jax.experimental.pallas import tpu_sc as plsc

N = 10000
E = 320000
F = 128
BN = 1000

NC, NS = 2, 16
NW = NC * NS
NPAD = 10240
NPT = NPAD // NS
CB = 128
EROWS = 2560
EPAD = EROWS * CB
AGG_ROWS = EROWS // NW
DEG_ROWS = EROWS // NS
DEG_K = 8


def _rsqrt_newton(d):
    i = lax.bitcast_convert_type(d, jnp.int32)
    i = jnp.int32(0x5F3759DF) - lax.shift_right_logical(i, 1)
    y = lax.bitcast_convert_type(i, jnp.float32)
    for _ in range(3):
        y = y * (1.5 - 0.5 * d * y * y)
    return y


def _sc_body(src_hbm, dst_hbm, ew_hbm, x_hbm, aggp_hbm, dis_hbm, *sc):
    dstD = sc[0:8]
    ewD = sc[8:16]
    srcv = sc[16:20]
    dstv = sc[20:24]
    ewv = sc[24:28]
    disw = sc[28:32]
    rows = sc[32:34]
    disv = sc[34]
    agg_sh, deg_sh, dis_sh = sc[35], sc[36], sc[37]
    semDl = sc[38:46]
    semDs = sc[46:54]
    semAl = sc[54:58]
    semAd = sc[58:62]
    sem_g = sc[62:64]
    sem_s = sc[64:66]

    cid = lax.axis_index("c")
    sid = lax.axis_index("s")
    wid = cid * NS + sid

    scope = jax.named_scope("sc_zero")
    scope.__enter__()
    def zrow(r, _):
        for j in range(F // 16):
            rows[0][r, pl.ds(j * 16, 16)] = jnp.zeros((16,), jnp.float32)
        return 0
    lax.fori_loop(0, CB, zrow, 0)
    zeros_out = [pltpu.async_copy(rows[0], agg_sh.at[pl.ds(sid * NPT + q * CB, CB)], sem_g[0])
                 for q in range(NPT // CB)]
    def zdeg(r, _):
        disv[pl.ds(r * 16, 16)] = jnp.zeros((16,), jnp.float32)
        return 0
    lax.fori_loop(0, NPT // 16, zdeg, 0)
    pltpu.sync_copy(disv, deg_sh.at[pl.ds(sid * NPT, NPT)])
    plsc.subcore_barrier()
    scope.__exit__(None, None, None)

    scope = jax.named_scope("sc_deg")
    scope.__enter__()
    DB = 8
    dbase = sid * DEG_ROWS
    def dlin(b, cr):
        pltpu.async_copy(dst_hbm.at[cr], dstD[b], semDl[b])
        pltpu.async_copy(ew_hbm.at[cr], ewD[b], semDl[b])
    def dlin_wait(b):
        pltpu.make_async_copy(dst_hbm.at[0], dstD[b], semDl[b]).wait()
        pltpu.make_async_copy(ew_hbm.at[0], ewD[b], semDl[b]).wait()
    for b in range(DB):
        dlin(b, dbase + b)
    def deg_loop(g, _):
        for b in range(DB):
            dlin_wait(b)
            pltpu.async_copy(ewD[b], deg_sh.at[dstD[b]], semDs[b], add=True)
        for b in range(DB):
            pltpu.make_async_copy(ewD[b], deg_sh.at[pl.ds(0, CB)], semDs[b]).wait()
            @pl.when(g < DEG_ROWS // DB - 1)
            def _():
                dlin(b, dbase + (g + 1) * DB + b)
        return 0
    lax.fori_loop(0, DEG_ROWS // DB, deg_loop, 0)
    plsc.subcore_barrier()
    scope.__exit__(None, None, None)

    scope = jax.named_scope("sc_dis")
    scope.__enter__()
    for z in zeros_out:
        z.wait()
    pltpu.sync_copy(deg_sh.at[pl.ds(sid * NPT, NPT)], disv)
    def dis_loop(r, _):
        d = disv[pl.ds(r * 16, 16)] + 1.0
        disv[pl.ds(r * 16, 16)] = _rsqrt_newton(d)
        return 0
    lax.fori_loop(0, NPT // 16, dis_loop, 0)
    pltpu.sync_copy(disv, dis_sh.at[pl.ds(sid * NPT, NPT)])
    @pl.when(cid == 0)
    def _():
        pltpu.sync_copy(disv, dis_hbm.at[pl.ds(sid * NPT, NPT)])
    plsc.subcore_barrier()
    scope.__exit__(None, None, None)

    scope = jax.named_scope("sc_agg")
    scope.__enter__()
    base = wid * AGG_ROWS
    GRP = AGG_ROWS // 4

    def alin(s, cr):
        pltpu.async_copy(src_hbm.at[cr], srcv[s], semAl[s])
        pltpu.async_copy(dst_hbm.at[cr], dstv[s], semAl[s])
        pltpu.async_copy(ew_hbm.at[cr], ewv[s], semAl[s])
    def stageA(s, r, cr):
        pltpu.make_async_copy(src_hbm.at[0], srcv[s], semAl[s]).wait()
        pltpu.make_async_copy(dst_hbm.at[0], dstv[s], semAl[s]).wait()
        pltpu.make_async_copy(ew_hbm.at[0], ewv[s], semAl[s]).wait()
        pltpu.async_copy(x_hbm.at[srcv[s]], rows[r], sem_g[r])
        pltpu.async_copy(dis_sh.at[srcv[s]], disw[s], semAd[s])
    def scale(r, s):
        def grp(j, _):
            s16 = ewv[s][pl.ds(j * 16, 16)] * disw[s][pl.ds(j * 16, 16)]
            for i in range(16):
                sc_ = s16[i]
                e = j * 16 + i
                for f_ in range(F // 16):
                    rows[r][e, pl.ds(f_ * 16, 16)] = rows[r][e, pl.ds(f_ * 16, 16)] * sc_
            return 0
        lax.fori_loop(0, CB // 16, grp, 0)
    def stageB(s, r):
        pltpu.make_async_copy(x_hbm.at[pl.ds(0, CB)], rows[r], sem_g[r]).wait()
        pltpu.make_async_copy(dis_sh.at[pl.ds(0, CB)], disw[s], semAd[s]).wait()
        scale(r, s)
        pltpu.async_copy(rows[r], agg_sh.at[dstv[s]], sem_s[r], add=True)
    def stageC(r):
        pltpu.make_async_copy(rows[r], agg_sh.at[pl.ds(0, CB)], sem_s[r]).wait()

    for s in range(4):
        alin(s, base + s)
    stageA(0, 0, base)
    stageA(1, 1, base + 1)
    def p2_loop(k, _):
        c = base + 4 * k
        more = k < GRP - 1
        stageB(0, 0)
        stageC(0)
        stageA(2, 0, c + 2)
        stageB(1, 1)
        stageC(1)
        stageA(3, 1, c + 3)
        @pl.when(more)
        def _():
            alin(0, c + 4)
        stageB(2, 0)
        stageC(0)
        @pl.when(more)
        def _():
            alin(1, c + 5)
            stageA(0, 0, c + 4)
        stageB(3, 1)
        stageC(1)
        @pl.when(more)
        def _():
            alin(2, c + 6)
            stageA(1, 1, c + 5)
            alin(3, c + 7)
        return 0
    lax.fori_loop(0, GRP, p2_loop, 0)
    plsc.subcore_barrier()
    scope.__exit__(None, None, None)

    scope = jax.named_scope("sc_out")
    scope.__enter__()
    outs = []
    for q in range(NPT // CB):
        off = sid * NPT + q * CB
        r = q % 2
        if q >= 2:
            outs[q - 2].wait()
        pltpu.sync_copy(agg_sh.at[pl.ds(off, CB)], rows[r])
        outs.append(pltpu.async_copy(rows[r], aggp_hbm.at[cid].at[pl.ds(off, CB)], sem_g[r]))
    outs[-2].wait()
    outs[-1].wait()
    scope.__exit__(None, None, None)


def _sparse_stage(src2, dst2, ew2, x):
    mesh = plsc.VectorSubcoreMesh(core_axis_name="c", subcore_axis_name="s")
    scratch = (
        [pltpu.VMEM((CB,), jnp.int32) for _ in range(8)] +
        [pltpu.VMEM((CB,), jnp.float32) for _ in range(8)] +
        [pltpu.VMEM((CB,), jnp.int32) for _ in range(4)] +
        [pltpu.VMEM((CB,), jnp.int32) for _ in range(4)] +
        [pltpu.VMEM((CB,), jnp.float32) for _ in range(4)] +
        [pltpu.VMEM((CB,), jnp.float32) for _ in range(4)] +
        [pltpu.VMEM((CB, F), jnp.float32) for _ in range(2)] +
        [pltpu.VMEM((NPT,), jnp.float32)] +
        [pltpu.VMEM_SHARED((NPAD, F), jnp.float32),
         pltpu.VMEM_SHARED((NPAD,), jnp.float32),
         pltpu.VMEM_SHARED((NPAD,), jnp.float32)] +
        [pltpu.SemaphoreType.DMA] * 28
    )
    f = pl.kernel(
        _sc_body,
        out_type=[jax.ShapeDtypeStruct((NC, NPAD, F), jnp.float32),
                  jax.ShapeDtypeStruct((NPAD,), jnp.float32)],
        mesh=mesh,
        scratch_types=scratch,
        compiler_params=pltpu.CompilerParams(needs_layout_passes=False),
    )
    return f(src2, dst2, ew2, x)


def _dense_body(dis_ref, x_ref, h_ref, agg_ref,
                wz_ref, bz_ref, wr_ref, br_ref, wh_ref, bh_ref,
                wlz_ref, blz_ref, wlr_ref, blr_ref, wlh_ref, blh_ref,
                wout_ref, bout_ref, y_ref, hn_ref):
    d = dis_ref[:]
    x = x_ref[:]
    h = h_ref[:]
    p = d * (agg_ref[0] + agg_ref[1]) + (d * d) * x

    def mm(a, b):
        return jax.lax.dot_general(a, b, (((1,), (0,)), ((), ())),
                                   preferred_element_type=jnp.float32)

    cz = mm(p, wz_ref[:]) + bz_ref[:]
    cr = mm(p, wr_ref[:]) + br_ref[:]
    ch = mm(p, wh_ref[:]) + bh_ref[:]

    z = jax.nn.sigmoid(mm(cz, wlz_ref[:F]) + mm(h, wlz_ref[F:]) + blz_ref[:])
    r = jax.nn.sigmoid(mm(cr, wlr_ref[:F]) + mm(h, wlr_ref[F:]) + blr_ref[:])
    ht = jnp.tanh(mm(ch, wlh_ref[:F]) + mm(h * r, wlh_ref[F:]) + blh_ref[:])
    hn = z * h + (1.0 - z) * ht
    hn_ref[:] = hn
    y_ref[:] = mm(jnp.maximum(hn, 0.0), wout_ref[:]) + bout_ref[:]


def _dense_stage(dis, x, h, agg, Wz, bz, Wr, br, Wh, bh,
                 Wlz, blz, Wlr, blr, Wlh, blh, Wout, bout):
    grid = (N // BN,)
    row_spec = pl.BlockSpec((BN, F), lambda i: (i, 0))
    full = pl.BlockSpec((2, BN, F), lambda i: (0, i, 0))
    w_spec = pl.BlockSpec((F, F), lambda i: (0, 0))
    wl_spec = pl.BlockSpec((2 * F, F), lambda i: (0, 0))
    b_spec = pl.BlockSpec((1, F), lambda i: (0, 0))
    return pl.pallas_call(
        _dense_body,
        grid=grid,
        in_specs=[
            pl.BlockSpec((BN, 1), lambda i: (i, 0)),
            row_spec, row_spec, full,
            w_spec, b_spec, w_spec, b_spec, w_spec, b_spec,
            wl_spec, b_spec, wl_spec, b_spec, wl_spec, b_spec,
            w_spec, b_spec,
        ],
        out_specs=[row_spec, row_spec],
        out_shape=[jax.ShapeDtypeStruct((N, F), jnp.float32),
                   jax.ShapeDtypeStruct((N, F), jnp.float32)],
    )(dis, x, h, agg,
      Wz, bz.reshape(1, F), Wr, br.reshape(1, F), Wh, bh.reshape(1, F),
      Wlz, blz.reshape(1, F), Wlr, blr.reshape(1, F), Wlh, blh.reshape(1, F),
      Wout, bout.reshape(1, F))


def kernel(g, node_feat, edge_weight, hidden_state, Wz, bz, Wr, br, Wh, bh,
           Wlz, blz, Wlr, blr, Wlh, blh, Wout, bout):
    src, dst = g[0], g[1]
    x = node_feat

    npadE = EPAD - E
    src2 = jnp.concatenate([src, jnp.zeros((npadE,), jnp.int32)]).reshape(EROWS, CB)
    dst2 = jnp.concatenate([dst, jnp.full((npadE,), NPAD - 1, jnp.int32)]).reshape(EROWS, CB)
    ew2 = jnp.concatenate([edge_weight, jnp.zeros((npadE,), jnp.float32)]).reshape(EROWS, CB)

    aggp, dis_pad = _sparse_stage(src2, dst2, ew2, x)
    agg2 = aggp[:, :N, :]
    dis = dis_pad[:N]

    y, hn = _dense_stage(dis.reshape(N, 1), x, hidden_state, agg2,
                         Wz, bz, Wr, br, Wh, bh,
                         Wlz, blz, Wlr, blr, Wlh, blh, Wout, bout)
    return (y, hn)
